# baseline (speedup 1.0000x reference)
import jax
import jax.numpy as jnp
from jax.experimental import pallas as pl

_BR = 12800
_N = 100000
_K = 118
_M = 16


def _packed_mm(x_ref, w_ref, o_ref):
    xv = x_ref[...].reshape(_BR // 8, 8, 128)
    acc = jnp.dot(xv[:, 0, :], w_ref[0],
                  preferred_element_type=jnp.float32)
    for a in range(1, 8):
        acc = acc + jnp.dot(xv[:, a, :], w_ref[a],
                            preferred_element_type=jnp.float32)
    o_ref[...] = acc


def kernel(atomic_numbers, atomic_energies):
    pad_id = jnp.eye(_K, 128, dtype=jnp.float32)
    xp = atomic_numbers @ pad_id

    wstack = jnp.zeros((8, 128, 128), jnp.float32)
    for a in range(8):
        wstack = wstack.at[a, :_K, _M * a:_M * (a + 1)].set(atomic_energies)

    grid = (_N + _BR - 1) // _BR
    out128 = pl.pallas_call(
        _packed_mm,
        grid=(grid,),
        in_specs=[
            pl.BlockSpec((_BR, 128), lambda i: (i, 0)),
            pl.BlockSpec((8, 128, 128), lambda i: (0, 0, 0)),
        ],
        out_specs=pl.BlockSpec((_BR // 8, 128), lambda i: (i, 0)),
        out_shape=jax.ShapeDtypeStruct((_N * _M // 128, 128), jnp.float32),
    )(xp, wstack)
    return out128.reshape(_N, _M)


# speedup vs baseline: 1.4296x; 1.4296x over previous
import jax
import jax.numpy as jnp
from jax.experimental import pallas as pl

_BR = 20000
_K = 118


def _mm_block(x_ref, w_ref, o_ref):
    o_ref[...] = jnp.dot(x_ref[...], w_ref[...],
                         preferred_element_type=jnp.float32)


def kernel(atomic_numbers, atomic_energies):
    n = atomic_numbers.shape[0]
    m = atomic_energies.shape[1]
    pad_id = jnp.eye(_K, 128, dtype=jnp.float32)
    xp = atomic_numbers @ pad_id
    wp = jnp.zeros((128, m), jnp.float32).at[:_K].set(atomic_energies)
    grid = n // _BR
    return pl.pallas_call(
        _mm_block,
        grid=(grid,),
        in_specs=[
            pl.BlockSpec((_BR, 128), lambda i: (i, 0)),
            pl.BlockSpec((128, m), lambda i: (0, 0)),
        ],
        out_specs=pl.BlockSpec((_BR, m), lambda i: (i, 0)),
        out_shape=jax.ShapeDtypeStruct((n, m), jnp.float32),
    )(xp, wp)
